# baseline (speedup 1.0000x reference)
import functools

import jax
import jax.numpy as jnp
import numpy as _np
from jax import lax
from jax.experimental import pallas as pl
from jax.experimental.pallas import tpu as pltpu
from jax.experimental.pallas import tpu_sc as plsc

BS, T, DIM, ND, NC = 1024, 16, 512, 365, 24
NDP, NCP, NCLS = 384, 32, 128
BLK = 256
NB = BS // BLK
SC_ROWS = BS // 32


def _dot(a, b):
    return jax.lax.dot_general(a, b, (((1,), (0,)), ((), ())),
                               preferred_element_type=jnp.float32)


def _dot_t(a, b):
    return jax.lax.dot_general(a, b, (((0,), (0,)), ((), ())),
                               preferred_element_type=jnp.float32)


def _prelu(x, a):
    return jnp.where(x >= 0, x, a * x)


SC_CH = 2
SC_NCH = SC_ROWS // SC_CH


SC_CH = 2
SC_NCH = SC_ROWS // SC_CH


def _sc_sum_body(frame_hbm, out_hbm, buf_a, buf_b, out_a, out_b,
                 sin_a, sin_b, sout_a, sout_b):
    cid = lax.axis_index("c")
    sid = lax.axis_index("s")
    wid = sid * 2 + cid
    base = wid * SC_ROWS

    def in_copy(g, buf, sem):
        return pltpu.make_async_copy(
            frame_hbm.at[pl.ds(base + g * SC_CH, SC_CH)], buf, sem)

    def out_copy(g, outb, sem):
        return pltpu.make_async_copy(
            outb, out_hbm.at[pl.ds(base + g * SC_CH, SC_CH)], sem)

    in_copy(0, buf_a, sin_a).start()
    in_copy(1, buf_b, sin_b).start()

    def reduce_chunk(g, buf, outb, sin, sout):
        in_copy(g, buf, sin).wait()
        for r in range(SC_CH):
            for j in range(DIM // 16):
                acc = buf[r, 0, pl.ds(j * 16, 16)]
                for t in range(1, T):
                    acc = acc + buf[r, t, pl.ds(j * 16, 16)]
                outb[r, pl.ds(j * 16, 16)] = acc

        @pl.when(g >= 2)
        def _():
            out_copy(g, outb, sout).wait()

        out_copy(g, outb, sout).start()

    def loop_body(g):
        reduce_chunk(g, buf_a, out_a, sin_a, sout_a)

        @pl.when(g + 2 < SC_NCH)
        def _():
            in_copy(g + 2, buf_a, sin_a).start()

        reduce_chunk(g + 1, buf_b, out_b, sin_b, sout_b)

        @pl.when(g + 3 < SC_NCH)
        def _():
            in_copy(g + 3, buf_b, sin_b).start()

    pl.loop(0, SC_NCH, step=2)(loop_body)

    out_copy(SC_NCH - 2, out_a, sout_a).wait()
    out_copy(SC_NCH - 1, out_b, sout_b).wait()


def _tc_body(video_ref, vd_ref, dv_ref, sub_ref, ev_ref, dc_ref, gt_ref,
             Wc2d_ref, bc2d_ref, ac2d_ref,
             Wv2d_ref, bv2d_ref, av2d_ref,
             Wd2v_ref, bd2v_ref, ad2v_ref,
             Wd2v2_ref, bd2v2_ref, ad2v2_ref,
             Wsa_ref, bsa_ref, qsa_ref,
             Wfc1_ref, Wfc2_ref, Wfc3_ref, bfc_ref,
             Wcls_ref, bcls_ref,
             loss_ref, idx_ref,
             video_s, d2v_s, acc_s, sw_s, c2d_s, aw2_s, lsum_s):
    p = pl.program_id(0)
    i = pl.program_id(1)

    @pl.when(jnp.logical_and(p == 0, i == 0))
    def _init():
        acc_s[:] = jnp.zeros_like(acc_s)
        sw_s[:] = _dot(sub_ref[:], Wd2v_ref[:])
        eW = _dot(ev_ref[:], Wc2d_ref[:])
        c2d_s[:] = _prelu(_dot(dc_ref[:], eW) + bc2d_ref[:], ac2d_ref[0, 0])

    @pl.when(p == 0)
    def _phase0():
        v = video_ref[:]
        video_s[pl.ds(i * BLK, BLK), :] = v
        vW = _dot(v, Wv2d_ref[:])
        acc_s[:] += _dot_t(dv_ref[:], vW)
        d2v_s[pl.ds(i * BLK, BLK), :] = _prelu(
            _dot(vd_ref[:], sw_s[:]) + bd2v_ref[:], ad2v_ref[0, 0])

    @pl.when(jnp.logical_and(p == 1, i == 0))
    def _transition():
        c2d = c2d_s[:]
        v2d = _prelu(acc_s[:] + bv2d_ref[:], av2d_ref[0, 0])
        qsa = qsa_ref[:]
        mask = jax.lax.broadcasted_iota(jnp.int32, (NDP, DIM // 4), 0) < ND
        hc = jnp.tanh(_dot(c2d, Wsa_ref[:]) + bsa_ref[:])
        hv = jnp.tanh(_dot(v2d, Wsa_ref[:]) + bsa_ref[:])
        sc = jnp.sum(jnp.where(mask, hc * qsa, 0.0)) / ND
        sv = jnp.sum(jnp.where(mask, hv * qsa, 0.0)) / ND
        m = jnp.maximum(sc, sv)
        e0, e1 = jnp.exp(sc - m), jnp.exp(sv - m)
        att = (e0 * c2d + e1 * v2d) / (e0 + e1)
        aw2_s[:] = _dot(att, Wd2v2_ref[:])
        lsum_s[0, 0] = 0.0

    @pl.when(p == 1)
    def _phase1():
        d2v2 = _prelu(_dot(vd_ref[:], aw2_s[:]) + bd2v2_ref[:],
                      ad2v2_ref[0, 0])
        vc = (_dot(d2v2, Wfc1_ref[:])
              + _dot(d2v_s[pl.ds(i * BLK, BLK), :], Wfc2_ref[:])
              + _dot(video_s[pl.ds(i * BLK, BLK), :], Wfc3_ref[:])
              + bfc_ref[:])
        preds = _dot(vc, Wcls_ref[:]) + bcls_ref[:]
        mx = jnp.max(preds, axis=1, keepdims=True)
        z = preds - mx
        lse = jnp.log(jnp.sum(jnp.exp(z), axis=1, keepdims=True))
        cls_ids = jax.lax.broadcasted_iota(jnp.int32, (BLK, NCLS), 1)
        z_gt = jnp.sum(jnp.where(cls_ids == gt_ref[:], z, 0.0), axis=1,
                       keepdims=True)
        lsum_s[0, 0] += jnp.sum(z_gt - lse)
        idx_ref[:] = jnp.min(jnp.where(preds == mx, cls_ids, NCLS), axis=1,
                             keepdims=True)

    @pl.when(jnp.logical_and(p == 1, i == NB - 1))
    def _final():
        loss_ref[:] = jnp.full((1, 1), -1.0 / BS) * lsum_s[0, 0]


@functools.partial(jax.jit, static_argnames=())
def kernel(frame_emb, cd_adj, dc_adj, vd_adj, dv_adj, subevent, event,
           logit_scale, ground_truth, W_c2d, b_c2d, a_c2d, W_v2d, b_v2d,
           a_v2d, W_d2v, b_d2v, a_d2v, W_d2v2, b_d2v2, a_d2v2, W_sa, b_sa,
           q_sa, W_fc, b_fc, W_cls, b_cls):
    del cd_adj, logit_scale

    f32 = jnp.float32

    video_sum = pl.kernel(
        _sc_sum_body,
        mesh=plsc.VectorSubcoreMesh(core_axis_name="c",
                                    subcore_axis_name="s"),
        out_type=jax.ShapeDtypeStruct((BS, DIM), f32),
        scratch_types=[
            pltpu.VMEM((SC_CH, T, DIM), f32),
            pltpu.VMEM((SC_CH, T, DIM), f32),
            pltpu.VMEM((SC_CH, DIM), f32),
            pltpu.VMEM((SC_CH, DIM), f32),
            pltpu.SemaphoreType.DMA,
            pltpu.SemaphoreType.DMA,
            pltpu.SemaphoreType.DMA,
            pltpu.SemaphoreType.DMA,
        ],
    )(frame_emb)

    pad_nd = NDP - ND
    vd_p = jnp.pad(vd_adj, ((0, 0), (0, pad_nd)))
    dv_p = jnp.pad(dv_adj, ((0, 0), (0, pad_nd)))
    sub_p = jnp.pad(subevent, ((0, pad_nd), (0, 0)))
    dc_p = jnp.pad(dc_adj, ((0, pad_nd), (0, NCP - NC)))
    ev_p = jnp.pad(event, ((0, NCP - NC), (0, 0)))
    Wcls_p = jnp.pad(W_cls, ((0, 0), (0, NCLS - NC)))
    bcls_p = jnp.concatenate(
        [b_cls, jnp.full((NCLS - NC,), -1e30, f32)]).reshape(1, NCLS)
    Wfc1, Wfc2 = W_fc[:DIM], W_fc[DIM:2 * DIM]
    Wfc3 = W_fc[2 * DIM:] * (1.0 / T)
    Wv2d_s = W_v2d * (1.0 / T)
    gt2 = ground_truth.reshape(BS, 1)
    s = lambda x: x.reshape(1, 1)
    r = lambda x: x.reshape(1, -1)

    def full_spec(shape):
        nd = len(shape)
        return pl.BlockSpec(shape, lambda p, i, _n=nd: (0,) * _n)

    loss2, idx = pl.pallas_call(
        _tc_body,
        grid=(2, NB),
        in_specs=[
            pl.BlockSpec((BLK, DIM),
                         lambda p, i: (i * (1 - p) + (NB - 1) * p, 0)),
            pl.BlockSpec((BLK, NDP), lambda p, i: (i, 0)),
            pl.BlockSpec((BLK, NDP),
                         lambda p, i: (i * (1 - p) + (NB - 1) * p, 0)),
            full_spec((NDP, DIM)),
            full_spec((NCP, DIM)),
            full_spec((NDP, NCP)),
            pl.BlockSpec((BLK, 1), lambda p, i: (i, 0)),
            full_spec((DIM, DIM)), full_spec((1, DIM)), full_spec((1, 1)),
            full_spec((DIM, DIM)), full_spec((1, DIM)), full_spec((1, 1)),
            full_spec((DIM, DIM)), full_spec((1, DIM)), full_spec((1, 1)),
            full_spec((DIM, DIM)), full_spec((1, DIM)), full_spec((1, 1)),
            full_spec((DIM, DIM // 4)), full_spec((1, DIM // 4)),
            full_spec((1, DIM // 4)),
            full_spec((DIM, DIM)), full_spec((DIM, DIM)),
            full_spec((DIM, DIM)), full_spec((1, DIM)),
            full_spec((DIM, NCLS)), full_spec((1, NCLS)),
        ],
        out_specs=(
            pl.BlockSpec((1, 1), lambda p, i: (0, 0)),
            pl.BlockSpec((BLK, 1), lambda p, i: (i, 0)),
        ),
        out_shape=(jax.ShapeDtypeStruct((1, 1), f32),
                   jax.ShapeDtypeStruct((BS, 1), jnp.int32)),
        scratch_shapes=[
            pltpu.VMEM((BS, DIM), f32),
            pltpu.VMEM((BS, DIM), f32),
            pltpu.VMEM((NDP, DIM), f32),
            pltpu.VMEM((NDP, DIM), f32),
            pltpu.VMEM((NDP, DIM), f32),
            pltpu.VMEM((NDP, DIM), f32),
            pltpu.SMEM((1, 1), f32),
        ],
    )(video_sum, vd_p, dv_p, sub_p, ev_p, dc_p, gt2,
      W_c2d, r(b_c2d), s(a_c2d),
      Wv2d_s, r(b_v2d), s(a_v2d),
      W_d2v, r(b_d2v), s(a_d2v),
      W_d2v2, r(b_d2v2), s(a_d2v2),
      W_sa, b_sa, q_sa,
      Wfc1, Wfc2, Wfc3, r(b_fc),
      Wcls_p, bcls_p)

    return loss2[0, 0], idx


# speedup vs baseline: 2.0862x; 2.0862x over previous
import functools

import jax
import jax.numpy as jnp
from jax.experimental import pallas as pl
from jax.experimental.pallas import tpu as pltpu

BS, T, DIM, ND, NC = 1024, 16, 512, 365, 24
BLK = 256
NB = BS // BLK


def _dot(a, b):
    return jax.lax.dot_general(a, b, (((1,), (0,)), ((), ())),
                               preferred_element_type=jnp.float32)


def _dot_t(a, b):
    return jax.lax.dot_general(a, b, (((0,), (0,)), ((), ())),
                               preferred_element_type=jnp.float32)


def _prelu(x, a):
    return jnp.where(x >= 0, x, a * x)


def _body(frame_ref, vd_ref, dv_ref, sub_ref, ev_ref, dc_ref, gt_ref,
          Wc2d_ref, bc2d_ref, ac2d_ref,
          Wv2d_ref, bv2d_ref, av2d_ref,
          Wd2v_ref, bd2v_ref, ad2v_ref,
          Wd2v2_ref, bd2v2_ref, ad2v2_ref,
          Wsa_ref, bsa_ref, qsa_ref,
          Wfc_ref, bfc_ref,
          Wcls_ref, bcls_ref,
          loss_ref, idx_ref,
          video_s, d2v_s, acc_s, sw_s, c2d_s, aw2_s, lsum_s):
    p = pl.program_id(0)
    i = pl.program_id(1)

    @pl.when(jnp.logical_and(p == 0, i == 0))
    def _init():
        acc_s[:] = jnp.zeros_like(acc_s)
        sw_s[:] = _dot(sub_ref[:], Wd2v_ref[:])
        eW = _dot(ev_ref[:], Wc2d_ref[:])
        c2d_s[:] = _prelu(_dot(dc_ref[:], eW) + bc2d_ref[:], ac2d_ref[0, 0])

    @pl.when(p == 0)
    def _phase0():
        v = jnp.mean(frame_ref[:], axis=1)
        video_s[pl.ds(i * BLK, BLK), :] = v
        vW = _dot(v, Wv2d_ref[:])
        acc_s[:] += _dot_t(dv_ref[:], vW)
        d2v_s[pl.ds(i * BLK, BLK), :] = _prelu(
            _dot(vd_ref[:], sw_s[:]) + bd2v_ref[:], ad2v_ref[0, 0])

    @pl.when(jnp.logical_and(p == 1, i == 0))
    def _transition():
        c2d = c2d_s[:]
        v2d = _prelu(acc_s[:] + bv2d_ref[:], av2d_ref[0, 0])
        qsa = qsa_ref[:]
        hc = jnp.tanh(_dot(c2d, Wsa_ref[:]) + bsa_ref[:])
        hv = jnp.tanh(_dot(v2d, Wsa_ref[:]) + bsa_ref[:])
        sc = jnp.sum(hc * qsa) / ND
        sv = jnp.sum(hv * qsa) / ND
        m = jnp.maximum(sc, sv)
        e0, e1 = jnp.exp(sc - m), jnp.exp(sv - m)
        att = (e0 * c2d + e1 * v2d) / (e0 + e1)
        aw2_s[:] = _dot(att, Wd2v2_ref[:])
        lsum_s[0, 0] = 0.0

    @pl.when(p == 1)
    def _phase1():
        d2v2 = _prelu(_dot(vd_ref[:], aw2_s[:]) + bd2v2_ref[:],
                      ad2v2_ref[0, 0])
        vc = (_dot(d2v2, Wfc_ref[0:DIM, :])
              + _dot(d2v_s[pl.ds(i * BLK, BLK), :], Wfc_ref[DIM:2 * DIM, :])
              + _dot(video_s[pl.ds(i * BLK, BLK), :],
                     Wfc_ref[2 * DIM:3 * DIM, :])
              + bfc_ref[:])
        preds = _dot(vc, Wcls_ref[:]) + bcls_ref[:]
        mx = jnp.max(preds, axis=1, keepdims=True)
        z = preds - mx
        lse = jnp.log(jnp.sum(jnp.exp(z), axis=1, keepdims=True))
        cls_ids = jax.lax.broadcasted_iota(jnp.int32, (BLK, NC), 1)
        z_gt = jnp.sum(jnp.where(cls_ids == gt_ref[:], z, 0.0), axis=1,
                       keepdims=True)
        lsum_s[0, 0] += jnp.sum(z_gt - lse)
        idx_ref[:] = jnp.min(jnp.where(preds == mx, cls_ids, NC), axis=1,
                             keepdims=True)

    @pl.when(jnp.logical_and(p == 1, i == NB - 1))
    def _final():
        loss_ref[:] = jnp.full((1, 1), -1.0 / BS) * lsum_s[0, 0]


@functools.partial(jax.jit, static_argnames=())
def kernel(frame_emb, cd_adj, dc_adj, vd_adj, dv_adj, subevent, event,
           logit_scale, ground_truth, W_c2d, b_c2d, a_c2d, W_v2d, b_v2d,
           a_v2d, W_d2v, b_d2v, a_d2v, W_d2v2, b_d2v2, a_d2v2, W_sa, b_sa,
           q_sa, W_fc, b_fc, W_cls, b_cls):
    del cd_adj, logit_scale

    f32 = jnp.float32
    gt2 = ground_truth.reshape(BS, 1)
    s = lambda x: x.reshape(1, 1)
    r = lambda x: x.reshape(1, -1)

    def full_spec(shape):
        nd = len(shape)
        return pl.BlockSpec(shape, lambda p, i, _n=nd: (0,) * _n)

    loss2, idx = pl.pallas_call(
        _body,
        grid=(2, NB),
        in_specs=[
            pl.BlockSpec((BLK, T, DIM),
                         lambda p, i: (i * (1 - p) + (NB - 1) * p, 0, 0)),
            pl.BlockSpec((BLK, ND), lambda p, i: (i, 0)),
            pl.BlockSpec((BLK, ND),
                         lambda p, i: (i * (1 - p) + (NB - 1) * p, 0)),
            full_spec((ND, DIM)),
            full_spec((NC, DIM)),
            full_spec((ND, NC)),
            pl.BlockSpec((BLK, 1), lambda p, i: (i, 0)),
            full_spec((DIM, DIM)), full_spec((1, DIM)), full_spec((1, 1)),
            full_spec((DIM, DIM)), full_spec((1, DIM)), full_spec((1, 1)),
            full_spec((DIM, DIM)), full_spec((1, DIM)), full_spec((1, 1)),
            full_spec((DIM, DIM)), full_spec((1, DIM)), full_spec((1, 1)),
            full_spec((DIM, DIM // 4)), full_spec((1, DIM // 4)),
            full_spec((1, DIM // 4)),
            full_spec((3 * DIM, DIM)), full_spec((1, DIM)),
            full_spec((DIM, NC)), full_spec((1, NC)),
        ],
        out_specs=(
            pl.BlockSpec((1, 1), lambda p, i: (0, 0)),
            pl.BlockSpec((BLK, 1), lambda p, i: (i, 0)),
        ),
        out_shape=(jax.ShapeDtypeStruct((1, 1), f32),
                   jax.ShapeDtypeStruct((BS, 1), jnp.int32)),
        scratch_shapes=[
            pltpu.VMEM((BS, DIM), f32),
            pltpu.VMEM((BS, DIM), f32),
            pltpu.VMEM((ND, DIM), f32),
            pltpu.VMEM((ND, DIM), f32),
            pltpu.VMEM((ND, DIM), f32),
            pltpu.VMEM((ND, DIM), f32),
            pltpu.SMEM((1, 1), f32),
        ],
    )(frame_emb, vd_adj, dv_adj, subevent, event, dc_adj, gt2,
      W_c2d, r(b_c2d), s(a_c2d),
      W_v2d, r(b_v2d), s(a_v2d),
      W_d2v, r(b_d2v), s(a_d2v),
      W_d2v2, r(b_d2v2), s(a_d2v2),
      W_sa, b_sa, q_sa,
      W_fc, r(b_fc),
      W_cls, r(b_cls))

    return loss2[0, 0], idx
